# baseline (speedup 1.0000x reference)
import functools

import jax
import jax.numpy as jnp
from jax import lax
from jax.experimental import pallas as pl
from jax.experimental.pallas import tpu as pltpu
from jax.experimental.pallas import tpu_sc as plsc

B = 16384
D = 128
NC = 2
NS = 16
NW = NC * NS
B_PER_W = B // NW
CHUNK = 128
NCH = B_PER_W // CHUNK
LANES = 16
DG = D // LANES


def _body(tri_hbm, ent_hbm, rel_hbm, out_hbm,
          slab_v, hidx_v, ridx_v, tidx_v,
          hbuf0, rbuf0, tbuf0, hbuf1, rbuf1, tbuf1,
          acc16, out_v, sem0, sem1):
    wid = lax.axis_index("s") * NC + lax.axis_index("c")
    base_row = wid * B_PER_W

    pltpu.sync_copy(tri_hbm.at[pl.ds(base_row * 3, B_PER_W * 3)], slab_v)
    iota = lax.iota(jnp.int32, LANES)

    def split(g, _):
        rows3 = (g * LANES + iota) * 3
        hidx_v[pl.ds(g * LANES, LANES)] = plsc.load_gather(slab_v, [rows3])
        ridx_v[pl.ds(g * LANES, LANES)] = plsc.load_gather(slab_v, [rows3 + 1])
        tidx_v[pl.ds(g * LANES, LANES)] = plsc.load_gather(slab_v, [rows3 + 2])
        return 0

    lax.fori_loop(0, B_PER_W // LANES, split, 0)

    bufs = [(hbuf0, rbuf0, tbuf0), (hbuf1, rbuf1, tbuf1)]
    sems = [sem0, sem1]

    def fire(j, slot):
        h, r, t = bufs[slot]
        s = sems[slot]
        pltpu.async_copy(ent_hbm.at[hidx_v.at[pl.ds(j * CHUNK, CHUNK)]], h, s)
        pltpu.async_copy(rel_hbm.at[ridx_v.at[pl.ds(j * CHUNK, CHUNK)]], r, s)
        pltpu.async_copy(ent_hbm.at[tidx_v.at[pl.ds(j * CHUNK, CHUNK)]], t, s)

    def wait(slot):
        h, r, t = bufs[slot]
        s = sems[slot]
        pltpu.make_async_copy(ent_hbm.at[hidx_v.at[pl.ds(0, CHUNK)]], h, s).wait()
        pltpu.make_async_copy(rel_hbm.at[ridx_v.at[pl.ds(0, CHUNK)]], r, s).wait()
        pltpu.make_async_copy(ent_hbm.at[tidx_v.at[pl.ds(0, CHUNK)]], t, s).wait()

    def compute(j, slot):
        hbuf, rbuf, tbuf = bufs[slot]

        def group(g, _):
            def row_fn(i, _):
                acc = (hbuf[i, pl.ds(0, LANES)]
                       * rbuf[i, pl.ds(0, LANES)]
                       * tbuf[i, pl.ds(0, LANES)])
                for dg in range(1, DG):
                    acc = acc + (hbuf[i, pl.ds(dg * LANES, LANES)]
                                 * rbuf[i, pl.ds(dg * LANES, LANES)]
                                 * tbuf[i, pl.ds(dg * LANES, LANES)])
                acc16[pl.ds((i - g * LANES) * LANES, LANES)] = acc
                return 0

            lax.fori_loop(g * LANES, (g + 1) * LANES, row_fn, 0)
            cols = iota * LANES
            score = plsc.load_gather(acc16, [cols])
            for l in range(1, LANES):
                score = score + plsc.load_gather(acc16, [cols + l])
            out_v[pl.ds(j * CHUNK + g * LANES, LANES)] = score
            return 0

        lax.fori_loop(0, CHUNK // LANES, group, 0)

    fire(0, 0)
    fire(1, 1)

    def ring(it, _):
        base = it * 2
        for b in range(2):
            j = base + b
            wait(b)
            compute(j, b)

            @pl.when(j + 2 < NCH)
            def _():
                fire(j + 2, b)
            return_val = 0
        return return_val

    lax.fori_loop(0, NCH // 2, ring, 0)

    pltpu.sync_copy(out_v, out_hbm.at[pl.ds(base_row, B_PER_W)])


@jax.jit
def _run(tri, ent, rel):
    mesh = plsc.VectorSubcoreMesh(core_axis_name="c", subcore_axis_name="s")
    return pl.kernel(
        _body,
        out_type=jax.ShapeDtypeStruct((B,), jnp.float32),
        mesh=mesh,
        compiler_params=pltpu.CompilerParams(needs_layout_passes=False),
        scratch_types=[
            pltpu.VMEM((B_PER_W * 3,), jnp.int32),
            pltpu.VMEM((B_PER_W,), jnp.int32),
            pltpu.VMEM((B_PER_W,), jnp.int32),
            pltpu.VMEM((B_PER_W,), jnp.int32),
            pltpu.VMEM((CHUNK, D), jnp.float32),
            pltpu.VMEM((CHUNK, D), jnp.float32),
            pltpu.VMEM((CHUNK, D), jnp.float32),
            pltpu.VMEM((CHUNK, D), jnp.float32),
            pltpu.VMEM((CHUNK, D), jnp.float32),
            pltpu.VMEM((CHUNK, D), jnp.float32),
            pltpu.VMEM((LANES * LANES,), jnp.float32),
            pltpu.VMEM((B_PER_W,), jnp.float32),
            pltpu.SemaphoreType.DMA,
            pltpu.SemaphoreType.DMA,
        ],
    )(tri, ent, rel)


def kernel(triplet_idx, entity_embedding, relation_embedding):
    return _run(triplet_idx.astype(jnp.int32).reshape(-1),
                entity_embedding, relation_embedding)
